# baseline (speedup 1.0000x reference)
import functools

import jax
import jax.numpy as jnp
from jax import lax
from jax.experimental import pallas as pl
from jax.experimental.pallas import tpu as pltpu
from jax.experimental.pallas import tpu_sc as plsc

B = 16384
N_FEAT = 128
EMB_DIM = 64
HIDDEN = 128
N_CLASSES = 64

_INFO = plsc.get_sparse_core_info()
_NC = _INFO.num_cores
_NS = _INFO.num_subcores
_NW = _NC * _NS
_BPW = B // _NW
_CHUNK = 128
_NCHUNK = _BPW // _CHUNK


def _gather_body(table_hbm, idx_hbm, out_hbm, idx_v, rows_v, sem):
    wid = lax.axis_index("s") * _NC + lax.axis_index("c")
    base = wid * _BPW
    pltpu.sync_copy(idx_hbm.at[wid], idx_v)
    copies = []
    for j in range(_NCHUNK):
        c = pltpu.make_async_copy(
            table_hbm.at[idx_v.at[j]],
            rows_v.at[pl.ds(j * _CHUNK, _CHUNK)],
            sem,
        )
        c.start()
        copies.append(c)
    for c in copies:
        c.wait()
    pltpu.sync_copy(rows_v, out_hbm.at[pl.ds(base, _BPW)])


@jax.jit
def _gather(emb, idx3):
    mesh = plsc.VectorSubcoreMesh(core_axis_name="c", subcore_axis_name="s")
    k = functools.partial(
        pl.kernel,
        mesh=mesh,
        out_type=jax.ShapeDtypeStruct((B, EMB_DIM), jnp.float32),
        scratch_types=[
            pltpu.VMEM((_NCHUNK, _CHUNK), jnp.int32),
            pltpu.VMEM((_BPW, EMB_DIM), jnp.float32),
            pltpu.SemaphoreType.DMA,
        ],
        compiler_params=pltpu.CompilerParams(use_tc_tiling_on_sc=False),
    )(_gather_body)
    return k(emb, idx3)


def _mlp_body(x_ref, e_ref, w1x_ref, w1e_ref, b1_ref, w2_ref, b2_ref, out_ref):
    hx = lax.dot_general(
        x_ref[...], w1x_ref[...], (((1,), (1,)), ((), ())),
        preferred_element_type=jnp.float32)
    he = lax.dot_general(
        e_ref[...], w1e_ref[...], (((1,), (1,)), ((), ())),
        preferred_element_type=jnp.float32)
    h = jnp.maximum(hx + he + b1_ref[...], 0.0)
    out_ref[...] = lax.dot_general(
        h, w2_ref[...], (((1,), (1,)), ((), ())),
        preferred_element_type=jnp.float32) + b2_ref[...]


_BLK = 2048


@jax.jit
def _mlp(x, e, w1x, w1e, b1, w2, b2):
    grid = (B // _BLK,)
    return pl.pallas_call(
        _mlp_body,
        grid=grid,
        in_specs=[
            pl.BlockSpec((_BLK, N_FEAT), lambda i: (i, 0)),
            pl.BlockSpec((_BLK, EMB_DIM), lambda i: (i, 0)),
            pl.BlockSpec((HIDDEN, N_FEAT), lambda i: (0, 0)),
            pl.BlockSpec((HIDDEN, EMB_DIM), lambda i: (0, 0)),
            pl.BlockSpec((1, HIDDEN), lambda i: (0, 0)),
            pl.BlockSpec((N_CLASSES, HIDDEN), lambda i: (0, 0)),
            pl.BlockSpec((1, N_CLASSES), lambda i: (0, 0)),
        ],
        out_specs=pl.BlockSpec((_BLK, N_CLASSES), lambda i: (i, 0)),
        out_shape=jax.ShapeDtypeStruct((B, N_CLASSES), jnp.float32),
    )(x, e, w1x, w1e, b1, w2, b2)


def kernel(x, node_idx, emb, W1, b1, W2, b2):
    idx3 = node_idx.astype(jnp.int32).reshape(_NW, _NCHUNK, _CHUNK)
    e = _gather(emb, idx3)
    w1x = W1[:, :N_FEAT]
    w1e = W1[:, N_FEAT:]
    return _mlp(x, e, w1x, w1e, b1.reshape(1, HIDDEN), W2,
                b2.reshape(1, N_CLASSES))


# speedup vs baseline: 1.6675x; 1.6675x over previous
import functools

import jax
import jax.numpy as jnp
from jax import lax
from jax.experimental import pallas as pl
from jax.experimental.pallas import tpu as pltpu
from jax.experimental.pallas import tpu_sc as plsc

B = 16384
N_FEAT = 128
EMB_DIM = 64
HIDDEN = 128
N_CLASSES = 64

_INFO = plsc.get_sparse_core_info()
_NC = _INFO.num_cores
_NS = _INFO.num_subcores
_NW = _NC * _NS
_BPW = B // _NW
_GRP = 64
_NGRP = _BPW // _GRP


def _gather_body(table_hbm, idx_hbm, out_hbm, idx_v, rows_v, sems):
    wid = lax.axis_index("s") * _NC + lax.axis_index("c")
    base = wid * _BPW
    pltpu.sync_copy(idx_hbm.at[wid], idx_v)

    def fire(g):
        sem = sems.at[lax.rem(g, 2)]
        for sub in range(_GRP // 16):
            off = g * _GRP + sub * 16
            v = idx_v[pl.ds(off, 16)]
            for l in range(16):
                pltpu.make_async_copy(
                    table_hbm.at[pl.ds(v[l], 1)],
                    rows_v.at[pl.ds(off + l, 1)],
                    sem,
                ).start()

    def drain(g):
        pltpu.make_async_copy(
            table_hbm.at[pl.ds(0, _GRP)],
            rows_v.at[pl.ds(g * _GRP, _GRP)],
            sems.at[lax.rem(g, 2)],
        ).wait()

    fire(0)

    def body(g, _):
        fire(g)
        drain(g - 1)
        return _

    lax.fori_loop(1, _NGRP, body, 0)
    drain(_NGRP - 1)
    pltpu.sync_copy(rows_v, out_hbm.at[pl.ds(base, _BPW)])


@jax.jit
def _gather(emb, idx2):
    mesh = plsc.VectorSubcoreMesh(core_axis_name="c", subcore_axis_name="s")
    k = functools.partial(
        pl.kernel,
        mesh=mesh,
        out_type=jax.ShapeDtypeStruct((B, EMB_DIM), jnp.float32),
        scratch_types=[
            pltpu.VMEM((_BPW,), jnp.int32),
            pltpu.VMEM((_BPW, EMB_DIM), jnp.float32),
            pltpu.SemaphoreType.DMA((2,)),
        ],
    )(_gather_body)
    return k(emb, idx2)


def _mlp_body(x_ref, e_ref, w1x_ref, w1e_ref, b1_ref, w2_ref, b2_ref, out_ref):
    hx = lax.dot_general(
        x_ref[...], w1x_ref[...], (((1,), (1,)), ((), ())),
        preferred_element_type=jnp.float32)
    he = lax.dot_general(
        e_ref[...], w1e_ref[...], (((1,), (1,)), ((), ())),
        preferred_element_type=jnp.float32)
    h = jnp.maximum(hx + he + b1_ref[...], 0.0)
    out_ref[...] = lax.dot_general(
        h, w2_ref[...], (((1,), (1,)), ((), ())),
        preferred_element_type=jnp.float32) + b2_ref[...]


_BLK = 2048


@jax.jit
def _mlp(x, e, w1x, w1e, b1, w2, b2):
    grid = (B // _BLK,)
    return pl.pallas_call(
        _mlp_body,
        grid=grid,
        in_specs=[
            pl.BlockSpec((_BLK, N_FEAT), lambda i: (i, 0)),
            pl.BlockSpec((_BLK, EMB_DIM), lambda i: (i, 0)),
            pl.BlockSpec((HIDDEN, N_FEAT), lambda i: (0, 0)),
            pl.BlockSpec((HIDDEN, EMB_DIM), lambda i: (0, 0)),
            pl.BlockSpec((1, HIDDEN), lambda i: (0, 0)),
            pl.BlockSpec((N_CLASSES, HIDDEN), lambda i: (0, 0)),
            pl.BlockSpec((1, N_CLASSES), lambda i: (0, 0)),
        ],
        out_specs=pl.BlockSpec((_BLK, N_CLASSES), lambda i: (i, 0)),
        out_shape=jax.ShapeDtypeStruct((B, N_CLASSES), jnp.float32),
    )(x, e, w1x, w1e, b1, w2, b2)


def kernel(x, node_idx, emb, W1, b1, W2, b2):
    idx2 = node_idx.astype(jnp.int32).reshape(_NW, _BPW)
    e = _gather(emb, idx2)
    w1x = W1[:, :N_FEAT]
    w1e = W1[:, N_FEAT:]
    return _mlp(x, e, w1x, w1e, b1.reshape(1, HIDDEN), W2,
                b2.reshape(1, N_CLASSES))


# speedup vs baseline: 1.7711x; 1.0621x over previous
import functools

import jax
import jax.numpy as jnp
from jax import lax
from jax.experimental import pallas as pl
from jax.experimental.pallas import tpu as pltpu
from jax.experimental.pallas import tpu_sc as plsc

B = 16384
N_FEAT = 128
N_NODES = 1000000
EMB_DIM = 64
HIDDEN = 128
N_CLASSES = 64

_NC = 2
_NS = 16
_NW = _NC * _NS
_BPW = B // _NW
_GRP = 64
_NGRP = _BPW // _GRP

_PACK = 128
_RBLK = 4096
_RBLK4 = _RBLK // 4
_NRB = (N_NODES + _RBLK - 1) // _RBLK
_NPACK = _NRB * _RBLK4


def _retile_body(embT_ref, out_ref):
    b = embT_ref[...].astype(jnp.bfloat16)
    u = pltpu.bitcast(b, jnp.uint32)
    ut = jnp.swapaxes(u, 0, 1)
    out_ref[...] = jnp.concatenate(
        [ut[q * _RBLK4:(q + 1) * _RBLK4] for q in range(4)], axis=1)


@jax.jit
def _retile(embT):
    return pl.pallas_call(
        _retile_body,
        grid=(_NRB,),
        in_specs=[pl.BlockSpec((EMB_DIM, _RBLK), lambda i: (0, i))],
        out_specs=pl.BlockSpec((_RBLK4, _PACK), lambda i: (i, 0)),
        out_shape=jax.ShapeDtypeStruct((_NPACK, _PACK), jnp.uint32),
    )(embT)


def _gather_body(table_hbm, idx_hbm, out_hbm, idx_v, rows_v, sems):
    wid = lax.axis_index("s") * _NC + lax.axis_index("c")
    base = wid * _BPW
    pltpu.sync_copy(idx_hbm.at[wid], idx_v)

    def fire(g):
        sem = sems.at[lax.rem(g, 2)]
        for sub in range(_GRP // 16):
            off = g * _GRP + sub * 16
            v = idx_v[pl.ds(off, 16)]
            for l in range(16):
                pltpu.make_async_copy(
                    table_hbm.at[pl.ds(v[l], 1)],
                    rows_v.at[pl.ds(off + l, 1)],
                    sem,
                ).start()

    def drain(g):
        pltpu.make_async_copy(
            table_hbm.at[pl.ds(0, _GRP)],
            rows_v.at[pl.ds(g * _GRP, _GRP)],
            sems.at[lax.rem(g, 2)],
        ).wait()

    fire(0)

    def body(g, _):
        fire(g)
        drain(g - 1)
        return _

    lax.fori_loop(1, _NGRP, body, 0)
    drain(_NGRP - 1)
    pltpu.sync_copy(rows_v, out_hbm.at[pl.ds(base, _BPW)])


@jax.jit
def _gather(table, idx2):
    mesh = plsc.VectorSubcoreMesh(core_axis_name="c", subcore_axis_name="s")
    k = functools.partial(
        pl.kernel,
        mesh=mesh,
        out_type=jax.ShapeDtypeStruct((B, _PACK), jnp.uint32),
        scratch_types=[
            pltpu.VMEM((_BPW,), jnp.int32),
            pltpu.VMEM((_BPW, _PACK), jnp.uint32),
            pltpu.SemaphoreType.DMA((2,)),
        ],
    )(_gather_body)
    return k(table, idx2)


def _mlp_body(x_ref, e4_ref, sel_ref, w1xT_ref, w1eT_lo_ref, w1eT_hi_ref,
              b1_ref, w2_ref, b2_ref, outT_ref):
    e4 = e4_ref[...]
    sel = sel_ref[...]
    e_u = jnp.where(
        sel < 2,
        jnp.where(sel == 0, e4[:, 0:32], e4[:, 32:64]),
        jnp.where(sel == 2, e4[:, 64:96], e4[:, 96:128]),
    )
    e_lo = lax.bitcast_convert_type(e_u << 16, jnp.float32)
    e_hi = lax.bitcast_convert_type(e_u & jnp.uint32(0xFFFF0000),
                                    jnp.float32)
    hx = lax.dot_general(
        x_ref[...], w1xT_ref[...], (((1,), (0,)), ((), ())),
        preferred_element_type=jnp.float32)
    he = lax.dot_general(
        e_lo, w1eT_lo_ref[...], (((1,), (0,)), ((), ())),
        preferred_element_type=jnp.float32)
    he = he + lax.dot_general(
        e_hi, w1eT_hi_ref[...], (((1,), (0,)), ((), ())),
        preferred_element_type=jnp.float32)
    h = jnp.maximum(hx + he + b1_ref[...], 0.0)
    outT_ref[...] = lax.dot_general(
        w2_ref[...], h, (((1,), (1,)), ((), ())),
        preferred_element_type=jnp.float32) + b2_ref[...]


_BLK = 2048


@jax.jit
def _mlp(x, e4, sel, w1xT, w1eT_lo, w1eT_hi, b1, w2, b2):
    grid = (B // _BLK,)
    return pl.pallas_call(
        _mlp_body,
        grid=grid,
        in_specs=[
            pl.BlockSpec((_BLK, N_FEAT), lambda i: (i, 0)),
            pl.BlockSpec((_BLK, _PACK), lambda i: (i, 0)),
            pl.BlockSpec((_BLK, 1), lambda i: (i, 0)),
            pl.BlockSpec((N_FEAT, HIDDEN), lambda i: (0, 0)),
            pl.BlockSpec((EMB_DIM // 2, HIDDEN), lambda i: (0, 0)),
            pl.BlockSpec((EMB_DIM // 2, HIDDEN), lambda i: (0, 0)),
            pl.BlockSpec((1, HIDDEN), lambda i: (0, 0)),
            pl.BlockSpec((N_CLASSES, HIDDEN), lambda i: (0, 0)),
            pl.BlockSpec((N_CLASSES, 1), lambda i: (0, 0)),
        ],
        out_specs=pl.BlockSpec((N_CLASSES, _BLK), lambda i: (0, i)),
        out_shape=jax.ShapeDtypeStruct((N_CLASSES, B), jnp.float32),
    )(x, e4, sel, w1xT, w1eT_lo, w1eT_hi, b1, w2, b2)


def kernel(x, node_idx, emb, W1, b1, W2, b2):
    idx = node_idx.astype(jnp.int32)
    local = idx % _RBLK
    row = (idx // _RBLK) * _RBLK4 + (local % _RBLK4)
    sel = (local // _RBLK4).reshape(B, 1)
    table = _retile(emb.T)
    e4 = _gather(table, row.reshape(_NW, _BPW))
    w1T = W1.T
    w1eT = w1T[N_FEAT:]
    outT = _mlp(x, e4, sel, w1T[:N_FEAT], w1eT[0::2], w1eT[1::2],
                b1.reshape(1, HIDDEN), W2, b2.reshape(N_CLASSES, 1))
    return outT.T
